# baseline (speedup 1.0000x reference)
import dataclasses
import functools

import jax
import jax.numpy as jnp
from jax import lax
from jax.experimental import pallas as pl
from jax.experimental.pallas import tpu as pltpu
from jax.experimental.pallas import tpu_sc as plsc

_B = 8192
_BR = 512


def _body(rows_ref, cols_ref, out_ref):
    xr = rows_ref[:, :]
    t1r = jnp.floor(xr)
    t2r = jnp.ceil(xr)
    i1r = t1r.astype(jnp.int32)
    i2r = jnp.minimum(t2r.astype(jnp.int32), _B - 1)

    xc = cols_ref[:, :]
    t1c = jnp.floor(xc)
    tc = jnp.where(jnp.ceil(xc) != t1c, xc - t1c, 0.0)
    w2 = 1.0 - tc

    col = jax.lax.broadcasted_iota(jnp.int32, (8, _B), 1)
    for g in range(_BR // 8):
        s = slice(g * 8, (g + 1) * 8)
        a = col == i1r[s, :]
        b = col == i2r[s, :]
        out_ref[s, :] = jnp.where(a, tc, 0.0) + jnp.where(b, w2, 0.0)


def _masked_write(idx_rows, idx_cols):
    rows = idx_rows.shape[0]
    return pl.pallas_call(
        _body,
        grid=(rows // _BR,),
        in_specs=[
            pl.BlockSpec((_BR, 1), lambda i: (i, 0)),
            pl.BlockSpec((1, _B), lambda i: (0, 0)),
        ],
        out_specs=pl.BlockSpec((_BR, _B), lambda i: (i, 0)),
        out_shape=jax.ShapeDtypeStruct((rows, _B), idx_rows.dtype),
        compiler_params=pltpu.CompilerParams(
            dimension_semantics=("arbitrary",),
        ),
    )(idx_rows, idx_cols)


_NW = 32
_RPW = _B // _NW
_CH = 4
_NBATCH = _RPW // 16


def _sc_compiler_params():
    cp = pltpu.CompilerParams()
    if "needs_layout_passes" in pltpu.CompilerParams.__dataclass_fields__:
        cp = dataclasses.replace(cp, needs_layout_passes=False)
    return cp


def _sc_impl(index):
    mesh = plsc.VectorSubcoreMesh(core_axis_name="c", subcore_axis_name="s")

    @functools.partial(
        pl.kernel,
        out_type=jax.ShapeDtypeStruct((_B, _B), jnp.float32),
        mesh=mesh,
        compiler_params=_sc_compiler_params(),
        scratch_types=[
            pltpu.VMEM((_B,), jnp.float32),
            pltpu.VMEM((_CH, _B), jnp.float32),
            pltpu.VMEM((_CH, _B), jnp.float32),
            pltpu.VMEM((2, 16), jnp.int32),
            pltpu.VMEM((2, 16), jnp.int32),
            pltpu.SemaphoreType.DMA,
            pltpu.SemaphoreType.DMA,
        ],
    )
    def k(idx_hbm, out_hbm, idx_v, buf0, buf1, s1, s2, sem0, sem1):
        wid = lax.axis_index("s") * 2 + lax.axis_index("c")
        base = wid * _RPW
        pltpu.sync_copy(idx_hbm, idx_v)
        zero16 = jnp.zeros((16,), jnp.float32)

        @pl.loop(0, _B // 16)
        def _(j):
            for r in range(_CH):
                buf0[r, pl.ds(j * 16, 16)] = zero16
                buf1[r, pl.ds(j * 16, 16)] = zero16

        lane = lax.iota(jnp.int32, 16)
        rl = lane & (_CH - 1)
        one_i = jnp.ones((16,), jnp.int32)
        zero_i = jnp.zeros((16,), jnp.int32)
        one_f = jnp.ones((16,), jnp.float32)
        cap = jnp.full((16,), _B - 1, jnp.int32)
        bufs = (buf0, buf1)
        sems = (sem0, sem1)

        @pl.loop(0, _NBATCH)
        def _(b):
            tok0 = base + b * 16
            x = idx_v[pl.ds(tok0, 16)]
            i1 = x.astype(jnp.int32)
            fr = x - i1.astype(jnp.float32)
            i2 = jnp.minimum(i1 + jnp.where(fr > 0, one_i, zero_i), cap)
            g1 = plsc.load_gather(idx_v, [i1])
            g2 = plsc.load_gather(idx_v, [i2])
            v1 = g1 - g1.astype(jnp.int32).astype(jnp.float32)
            v2 = 1.0 - (g2 - g2.astype(jnp.int32).astype(jnp.float32))
            eq = i1 == i2
            v1 = jnp.where(eq, one_f, v1)
            v2 = jnp.where(eq, one_f, v2)
            for c in range(4):
                bi = c % 2
                buf = bufs[bi]
                sem = sems[bi]
                dst = out_hbm.at[pl.ds(tok0 + c * _CH, _CH)]

                def _drain_and_clear(pc, buf=buf, sem=sem, dst=dst, bi=bi):
                    pltpu.make_async_copy(buf, dst, sem).wait()
                    pmask = (lane >> 2) == pc
                    plsc.store_scatter(buf, [rl, s1[bi, :]], zero16, mask=pmask)
                    plsc.store_scatter(buf, [rl, s2[bi, :]], zero16, mask=pmask)

                if c < 2:
                    @pl.when(b > 0)
                    def _():
                        _drain_and_clear(jnp.int32(c + 2))
                else:
                    _drain_and_clear(jnp.int32(c - 2))
                mask = (lane >> 2) == c
                plsc.store_scatter(buf, [rl, i1], v1, mask=mask)
                plsc.store_scatter(buf, [rl, i2], v2, mask=mask)
                s1[bi, :] = i1
                s2[bi, :] = i2
                pltpu.make_async_copy(buf, dst, sem).start()

        dst0 = out_hbm.at[pl.ds(base, _CH)]
        pltpu.make_async_copy(buf0, dst0, sem0).wait()
        pltpu.make_async_copy(buf1, dst0, sem1).wait()

    return k(index)


_GR = _B // 128
_FROWS = _B * _GR
_ZROWS = 256
_ZDMAS = _RPW * _GR // _ZROWS


def _sc_impl2(index):
    mesh = plsc.VectorSubcoreMesh(core_axis_name="c", subcore_axis_name="s")

    @functools.partial(
        pl.kernel,
        out_type=jax.ShapeDtypeStruct((_FROWS, 128), jnp.float32),
        mesh=mesh,
        compiler_params=_sc_compiler_params(),
        scratch_types=[
            pltpu.VMEM((_B,), jnp.float32),
            pltpu.VMEM((_ZROWS, 128), jnp.float32),
            pltpu.VMEM((_RPW, 128), jnp.float32),
            pltpu.VMEM((_RPW, 128), jnp.float32),
            pltpu.VMEM((_RPW,), jnp.int32),
            pltpu.VMEM((_RPW,), jnp.int32),
            pltpu.SemaphoreType.DMA,
            pltpu.SemaphoreType.DMA,
        ],
    )
    def k(idx_hbm, out_hbm, idx_v, zbuf, wa, wb, ra_v, rb_v, zsem, wsem):
        wid = lax.axis_index("s") * 2 + lax.axis_index("c")
        base = wid * _RPW
        fbase = base * _GR
        zero16 = jnp.zeros((16,), jnp.float32)

        @pl.loop(0, _ZROWS)
        def _(r):
            for j in range(8):
                zbuf[r, pl.ds(j * 16, 16)] = zero16

        @pl.loop(0, _ZDMAS)
        def _(i):
            dst = out_hbm.at[pl.ds(fbase + i * _ZROWS, _ZROWS)]
            pltpu.make_async_copy(zbuf, dst, zsem).start()

        pltpu.sync_copy(idx_hbm, idx_v)

        @pl.loop(0, _RPW)
        def _(r):
            for j in range(8):
                wa[r, pl.ds(j * 16, 16)] = zero16
                wb[r, pl.ds(j * 16, 16)] = zero16

        lane = lax.iota(jnp.int32, 16)
        one_i = jnp.ones((16,), jnp.int32)
        zero_i = jnp.zeros((16,), jnp.int32)
        one_f = jnp.ones((16,), jnp.float32)
        cap = jnp.full((16,), _B - 1, jnp.int32)

        @pl.loop(0, _NBATCH)
        def _(b):
            tok0 = base + b * 16
            x = idx_v[pl.ds(tok0, 16)]
            i1 = x.astype(jnp.int32)
            fr = x - i1.astype(jnp.float32)
            i2 = jnp.minimum(i1 + jnp.where(fr > 0, one_i, zero_i), cap)
            g1 = plsc.load_gather(idx_v, [i1])
            g2 = plsc.load_gather(idx_v, [i2])
            v1 = g1 - g1.astype(jnp.int32).astype(jnp.float32)
            v2 = 1.0 - (g2 - g2.astype(jnp.int32).astype(jnp.float32))
            eq = i1 == i2
            v1 = jnp.where(eq, one_f, v1)
            v2 = jnp.where(eq, one_f, v2)
            la = i1 & 127
            lb = i2 & 127
            sg = (i1 >> 7) == (i2 >> 7)
            jj = b * 16 + lane
            plsc.store_scatter(wa, [jj, la], v1)
            plsc.store_scatter(wa, [jj, lb], v2, mask=sg)
            plsc.store_scatter(wb, [jj, lb], v2)
            plsc.store_scatter(wb, [jj, la], v1, mask=sg)
            ra_v[pl.ds(b * 16, 16)] = (tok0 + lane) * _GR + (i1 >> 7)
            rb_v[pl.ds(b * 16, 16)] = (tok0 + lane) * _GR + (i2 >> 7)

        @pl.loop(0, _ZDMAS)
        def _(i):
            dst = out_hbm.at[pl.ds(fbase + i * _ZROWS, _ZROWS)]
            pltpu.make_async_copy(zbuf, dst, zsem).wait()

        @pl.loop(0, _NBATCH)
        def _(b):
            ra = ra_v[pl.ds(b * 16, 16)]
            rb = rb_v[pl.ds(b * 16, 16)]
            src_a = wa.at[pl.ds(b * 16, 16)]
            src_b = wb.at[pl.ds(b * 16, 16)]
            pltpu.make_async_copy(src_a, out_hbm.at[ra], wsem).start()
            pltpu.make_async_copy(src_b, out_hbm.at[rb], wsem).start()

        @pl.loop(0, _NBATCH)
        def _(b):
            ra = ra_v[pl.ds(b * 16, 16)]
            rb = rb_v[pl.ds(b * 16, 16)]
            src_a = wa.at[pl.ds(b * 16, 16)]
            src_b = wb.at[pl.ds(b * 16, 16)]
            pltpu.make_async_copy(src_a, out_hbm.at[ra], wsem).wait()
            pltpu.make_async_copy(src_b, out_hbm.at[rb], wsem).wait()

    return k(index).reshape(_B, _B)


def kernel(index):
    return _sc_impl2(index)


def _tc_kernel(index):
    idx_rows = index.reshape(_B, 1)
    idx_cols = index.reshape(1, _B)
    return _masked_write(idx_rows, idx_cols)


# speedup vs baseline: 1.4080x; 1.4080x over previous
import dataclasses
import functools

import jax
import jax.numpy as jnp
from jax import lax
from jax.experimental import pallas as pl
from jax.experimental.pallas import tpu as pltpu
from jax.experimental.pallas import tpu_sc as plsc

_B = 8192
_BR = 512


def _body(rows_ref, cols_ref, out_ref):
    xr = rows_ref[:, :]
    t1r = jnp.floor(xr)
    t2r = jnp.ceil(xr)
    i1r = t1r.astype(jnp.int32)
    i2r = jnp.minimum(t2r.astype(jnp.int32), _B - 1)

    xc = cols_ref[:, :]
    t1c = jnp.floor(xc)
    tc = jnp.where(jnp.ceil(xc) != t1c, xc - t1c, 0.0)
    w2 = 1.0 - tc

    col = jax.lax.broadcasted_iota(jnp.int32, (8, _B), 1)
    for g in range(_BR // 8):
        s = slice(g * 8, (g + 1) * 8)
        a = col == i1r[s, :]
        b = col == i2r[s, :]
        out_ref[s, :] = jnp.where(a, tc, 0.0) + jnp.where(b, w2, 0.0)


def _masked_write(idx_rows, idx_cols):
    rows = idx_rows.shape[0]
    return pl.pallas_call(
        _body,
        grid=(rows // _BR,),
        in_specs=[
            pl.BlockSpec((_BR, 1), lambda i: (i, 0)),
            pl.BlockSpec((1, _B), lambda i: (0, 0)),
        ],
        out_specs=pl.BlockSpec((_BR, _B), lambda i: (i, 0)),
        out_shape=jax.ShapeDtypeStruct((rows, _B), idx_rows.dtype),
        compiler_params=pltpu.CompilerParams(
            dimension_semantics=("arbitrary",),
        ),
    )(idx_rows, idx_cols)


_NW = 32
_RPW = _B // _NW
_CH = 4
_NBATCH = _RPW // 16


def _sc_compiler_params():
    cp = pltpu.CompilerParams()
    if "needs_layout_passes" in pltpu.CompilerParams.__dataclass_fields__:
        cp = dataclasses.replace(cp, needs_layout_passes=False)
    return cp


def _sc_impl(index):
    mesh = plsc.VectorSubcoreMesh(core_axis_name="c", subcore_axis_name="s")

    @functools.partial(
        pl.kernel,
        out_type=jax.ShapeDtypeStruct((_B, _B), jnp.float32),
        mesh=mesh,
        compiler_params=_sc_compiler_params(),
        scratch_types=[
            pltpu.VMEM((_B,), jnp.float32),
            pltpu.VMEM((_CH, _B), jnp.float32),
            pltpu.VMEM((_CH, _B), jnp.float32),
            pltpu.VMEM((2, 16), jnp.int32),
            pltpu.VMEM((2, 16), jnp.int32),
            pltpu.SemaphoreType.DMA,
            pltpu.SemaphoreType.DMA,
        ],
    )
    def k(idx_hbm, out_hbm, idx_v, buf0, buf1, s1, s2, sem0, sem1):
        wid = lax.axis_index("s") * 2 + lax.axis_index("c")
        base = wid * _RPW
        pltpu.sync_copy(idx_hbm, idx_v)
        zero16 = jnp.zeros((16,), jnp.float32)

        @pl.loop(0, _B // 16)
        def _(j):
            for r in range(_CH):
                buf0[r, pl.ds(j * 16, 16)] = zero16
                buf1[r, pl.ds(j * 16, 16)] = zero16

        lane = lax.iota(jnp.int32, 16)
        rl = lane & (_CH - 1)
        one_i = jnp.ones((16,), jnp.int32)
        zero_i = jnp.zeros((16,), jnp.int32)
        one_f = jnp.ones((16,), jnp.float32)
        cap = jnp.full((16,), _B - 1, jnp.int32)
        bufs = (buf0, buf1)
        sems = (sem0, sem1)

        @pl.loop(0, _NBATCH)
        def _(b):
            tok0 = base + b * 16
            x = idx_v[pl.ds(tok0, 16)]
            i1 = x.astype(jnp.int32)
            fr = x - i1.astype(jnp.float32)
            i2 = jnp.minimum(i1 + jnp.where(fr > 0, one_i, zero_i), cap)
            g1 = plsc.load_gather(idx_v, [i1])
            g2 = plsc.load_gather(idx_v, [i2])
            v1 = g1 - g1.astype(jnp.int32).astype(jnp.float32)
            v2 = 1.0 - (g2 - g2.astype(jnp.int32).astype(jnp.float32))
            eq = i1 == i2
            v1 = jnp.where(eq, one_f, v1)
            v2 = jnp.where(eq, one_f, v2)
            for c in range(4):
                bi = c % 2
                buf = bufs[bi]
                sem = sems[bi]
                dst = out_hbm.at[pl.ds(tok0 + c * _CH, _CH)]

                def _drain_and_clear(pc, buf=buf, sem=sem, dst=dst, bi=bi):
                    pltpu.make_async_copy(buf, dst, sem).wait()
                    pmask = (lane >> 2) == pc
                    plsc.store_scatter(buf, [rl, s1[bi, :]], zero16, mask=pmask)
                    plsc.store_scatter(buf, [rl, s2[bi, :]], zero16, mask=pmask)

                if c < 2:
                    @pl.when(b > 0)
                    def _():
                        _drain_and_clear(jnp.int32(c + 2))
                else:
                    _drain_and_clear(jnp.int32(c - 2))
                mask = (lane >> 2) == c
                plsc.store_scatter(buf, [rl, i1], v1, mask=mask)
                plsc.store_scatter(buf, [rl, i2], v2, mask=mask)
                s1[bi, :] = i1
                s2[bi, :] = i2
                pltpu.make_async_copy(buf, dst, sem).start()

        dst0 = out_hbm.at[pl.ds(base, _CH)]
        pltpu.make_async_copy(buf0, dst0, sem0).wait()
        pltpu.make_async_copy(buf1, dst0, sem1).wait()

    return k(index)


_GR = _B // 128
_FROWS = _B * _GR
_ZROWS = 256
_ZDMAS = _RPW * _GR // _ZROWS


def _sc_impl2(index):
    mesh = plsc.VectorSubcoreMesh(core_axis_name="c", subcore_axis_name="s")

    @functools.partial(
        pl.kernel,
        out_type=jax.ShapeDtypeStruct((_FROWS, 128), jnp.float32),
        mesh=mesh,
        compiler_params=_sc_compiler_params(),
        scratch_types=[
            pltpu.VMEM((_B,), jnp.float32),
            pltpu.VMEM((_ZROWS, 128), jnp.float32),
            pltpu.VMEM((_RPW, 128), jnp.float32),
            pltpu.VMEM((_RPW, 128), jnp.float32),
            pltpu.VMEM((_RPW,), jnp.int32),
            pltpu.VMEM((_RPW,), jnp.int32),
            pltpu.SemaphoreType.DMA,
            pltpu.SemaphoreType.DMA,
        ],
    )
    def k(idx_hbm, out_hbm, idx_v, zbuf, wa, wb, ra_v, rb_v, zsem, wsem):
        wid = lax.axis_index("s") * 2 + lax.axis_index("c")
        base = wid * _RPW
        fbase = base * _GR
        zero16 = jnp.zeros((16,), jnp.float32)

        @pl.loop(0, _ZROWS)
        def _(r):
            for j in range(8):
                zbuf[r, pl.ds(j * 16, 16)] = zero16

        @pl.loop(0, _ZDMAS)
        def _(i):
            dst = out_hbm.at[pl.ds(fbase + i * _ZROWS, _ZROWS)]
            pltpu.make_async_copy(zbuf, dst, zsem).start()

        pltpu.sync_copy(idx_hbm, idx_v)

        @pl.loop(0, _RPW)
        def _(r):
            for j in range(8):
                wa[r, pl.ds(j * 16, 16)] = zero16
                wb[r, pl.ds(j * 16, 16)] = zero16

        lane = lax.iota(jnp.int32, 16)
        one_i = jnp.ones((16,), jnp.int32)
        zero_i = jnp.zeros((16,), jnp.int32)
        one_f = jnp.ones((16,), jnp.float32)
        cap = jnp.full((16,), _B - 1, jnp.int32)

        @pl.loop(0, _NBATCH)
        def _(b):
            tok0 = base + b * 16
            x = idx_v[pl.ds(tok0, 16)]
            i1 = x.astype(jnp.int32)
            fr = x - i1.astype(jnp.float32)
            i2 = jnp.minimum(i1 + jnp.where(fr > 0, one_i, zero_i), cap)
            g1 = plsc.load_gather(idx_v, [i1])
            g2 = plsc.load_gather(idx_v, [i2])
            v1 = g1 - g1.astype(jnp.int32).astype(jnp.float32)
            v2 = 1.0 - (g2 - g2.astype(jnp.int32).astype(jnp.float32))
            eq = i1 == i2
            v1 = jnp.where(eq, one_f, v1)
            v2 = jnp.where(eq, one_f, v2)
            la = i1 & 127
            lb = i2 & 127
            sg = (i1 >> 7) == (i2 >> 7)
            jj = b * 16 + lane
            plsc.store_scatter(wa, [jj, la], v1)
            plsc.store_scatter(wa, [jj, lb], v2, mask=sg)
            plsc.store_scatter(wb, [jj, lb], v2)
            plsc.store_scatter(wb, [jj, la], v1, mask=sg)
            ra_v[pl.ds(b * 16, 16)] = (tok0 + lane) * _GR + (i1 >> 7)
            rb_v[pl.ds(b * 16, 16)] = (tok0 + lane) * _GR + (i2 >> 7)

        @pl.loop(0, _ZDMAS)
        def _(i):
            dst = out_hbm.at[pl.ds(fbase + i * _ZROWS, _ZROWS)]
            pltpu.make_async_copy(zbuf, dst, zsem).wait()

        @pl.loop(0, _NBATCH)
        def _(b):
            ra = ra_v[pl.ds(b * 16, 16)]
            rb = rb_v[pl.ds(b * 16, 16)]
            src_a = wa.at[pl.ds(b * 16, 16)]
            src_b = wb.at[pl.ds(b * 16, 16)]
            pltpu.make_async_copy(src_a, out_hbm.at[ra], wsem).start()
            pltpu.make_async_copy(src_b, out_hbm.at[rb], wsem).start()

        @pl.loop(0, _NBATCH)
        def _(b):
            ra = ra_v[pl.ds(b * 16, 16)]
            rb = rb_v[pl.ds(b * 16, 16)]
            src_a = wa.at[pl.ds(b * 16, 16)]
            src_b = wb.at[pl.ds(b * 16, 16)]
            pltpu.make_async_copy(src_a, out_hbm.at[ra], wsem).wait()
            pltpu.make_async_copy(src_b, out_hbm.at[rb], wsem).wait()

    return k(index).reshape(_B, _B)


_R0 = 4608
_SCROWS = _B - _R0
_RPW3 = _SCROWS // _NW
_NBATCH3 = _RPW3 // 16


def _sc_rows(index):
    mesh = plsc.VectorSubcoreMesh(core_axis_name="c", subcore_axis_name="s")

    @functools.partial(
        pl.kernel,
        out_type=jax.ShapeDtypeStruct((_SCROWS, _B), jnp.float32),
        mesh=mesh,
        compiler_params=_sc_compiler_params(),
        scratch_types=[
            pltpu.VMEM((_B,), jnp.float32),
            pltpu.VMEM((_CH, _B), jnp.float32),
            pltpu.VMEM((_CH, _B), jnp.float32),
            pltpu.VMEM((2, 16), jnp.int32),
            pltpu.VMEM((2, 16), jnp.int32),
            pltpu.SemaphoreType.DMA,
            pltpu.SemaphoreType.DMA,
        ],
    )
    def k(idx_hbm, out_hbm, idx_v, buf0, buf1, s1, s2, sem0, sem1):
        wid = lax.axis_index("s") * 2 + lax.axis_index("c")
        base = wid * _RPW3
        pltpu.sync_copy(idx_hbm, idx_v)
        zero16 = jnp.zeros((16,), jnp.float32)

        @pl.loop(0, _B // 16)
        def _(j):
            for r in range(_CH):
                buf0[r, pl.ds(j * 16, 16)] = zero16
                buf1[r, pl.ds(j * 16, 16)] = zero16

        lane = lax.iota(jnp.int32, 16)
        rl = lane & (_CH - 1)
        one_i = jnp.ones((16,), jnp.int32)
        zero_i = jnp.zeros((16,), jnp.int32)
        one_f = jnp.ones((16,), jnp.float32)
        cap = jnp.full((16,), _B - 1, jnp.int32)
        bufs = (buf0, buf1)
        sems = (sem0, sem1)

        @pl.loop(0, _NBATCH3)
        def _(b):
            row0 = base + b * 16
            x = idx_v[pl.ds(_R0 + row0, 16)]
            i1 = x.astype(jnp.int32)
            fr = x - i1.astype(jnp.float32)
            i2 = jnp.minimum(i1 + jnp.where(fr > 0, one_i, zero_i), cap)
            g1 = plsc.load_gather(idx_v, [i1])
            g2 = plsc.load_gather(idx_v, [i2])
            v1 = g1 - g1.astype(jnp.int32).astype(jnp.float32)
            v2 = 1.0 - (g2 - g2.astype(jnp.int32).astype(jnp.float32))
            eq = i1 == i2
            v1 = jnp.where(eq, one_f, v1)
            v2 = jnp.where(eq, one_f, v2)
            for c in range(4):
                bi = c % 2
                buf = bufs[bi]
                sem = sems[bi]
                dst = out_hbm.at[pl.ds(row0 + c * _CH, _CH)]

                def _drain_and_clear(pc, buf=buf, sem=sem, dst=dst, bi=bi):
                    pltpu.make_async_copy(buf, dst, sem).wait()
                    pmask = (lane >> 2) == pc
                    plsc.store_scatter(buf, [rl, s1[bi, :]], zero16, mask=pmask)
                    plsc.store_scatter(buf, [rl, s2[bi, :]], zero16, mask=pmask)

                if c < 2:
                    @pl.when(b > 0)
                    def _():
                        _drain_and_clear(jnp.int32(c + 2))
                else:
                    _drain_and_clear(jnp.int32(c - 2))
                mask = (lane >> 2) == c
                plsc.store_scatter(buf, [rl, i1], v1, mask=mask)
                plsc.store_scatter(buf, [rl, i2], v2, mask=mask)
                s1[bi, :] = i1
                s2[bi, :] = i2
                pltpu.make_async_copy(buf, dst, sem).start()

        dst0 = out_hbm.at[pl.ds(base, _CH)]
        pltpu.make_async_copy(buf0, dst0, sem0).wait()
        pltpu.make_async_copy(buf1, dst0, sem1).wait()

    return k(index)


def kernel(index):
    tc_part = _masked_write(index[:_R0].reshape(_R0, 1), index.reshape(1, _B))
    sc_part = _sc_rows(index)
    return jnp.concatenate([tc_part, sc_part], axis=0)


def _tc_kernel(index):
    idx_rows = index.reshape(_B, 1)
    idx_cols = index.reshape(1, _B)
    return _masked_write(idx_rows, idx_cols)


# speedup vs baseline: 3.6211x; 2.5718x over previous
import dataclasses
import functools

import jax
import jax.numpy as jnp
from jax import lax
from jax.experimental import pallas as pl
from jax.experimental.pallas import tpu as pltpu
from jax.experimental.pallas import tpu_sc as plsc

_B = 8192
_NW = 32
_RPW = _B // _NW
_CH = 4
_NBATCH = _RPW // 16


def _sc_compiler_params():
    cp = pltpu.CompilerParams()
    if "needs_layout_passes" in pltpu.CompilerParams.__dataclass_fields__:
        cp = dataclasses.replace(cp, needs_layout_passes=False)
    return cp


def _sc_bandwidthify(index):
    mesh = plsc.VectorSubcoreMesh(core_axis_name="c", subcore_axis_name="s")

    @functools.partial(
        pl.kernel,
        out_type=jax.ShapeDtypeStruct((_B, _B), jnp.float32),
        mesh=mesh,
        compiler_params=_sc_compiler_params(),
        scratch_types=[
            pltpu.VMEM((_B,), jnp.float32),
            pltpu.VMEM((_CH, _B), jnp.float32),
            pltpu.VMEM((_CH, _B), jnp.float32),
            pltpu.VMEM((2, 16), jnp.int32),
            pltpu.VMEM((2, 16), jnp.int32),
            pltpu.SemaphoreType.DMA,
            pltpu.SemaphoreType.DMA,
        ],
    )
    def k(idx_hbm, out_hbm, idx_v, buf0, buf1, s1, s2, sem0, sem1):
        wid = lax.axis_index("s") * 2 + lax.axis_index("c")
        base = wid * _RPW
        pltpu.sync_copy(idx_hbm, idx_v)
        zero16 = jnp.zeros((16,), jnp.float32)

        @pl.loop(0, _B // 16)
        def _(j):
            for r in range(_CH):
                buf0[r, pl.ds(j * 16, 16)] = zero16
                buf1[r, pl.ds(j * 16, 16)] = zero16

        lane = lax.iota(jnp.int32, 16)
        rl = lane & (_CH - 1)
        one_i = jnp.ones((16,), jnp.int32)
        zero_i = jnp.zeros((16,), jnp.int32)
        one_f = jnp.ones((16,), jnp.float32)
        cap = jnp.full((16,), _B - 1, jnp.int32)
        bufs = (buf0, buf1)
        sems = (sem0, sem1)

        @pl.loop(0, _NBATCH)
        def _(b):
            tok0 = base + b * 16
            x = idx_v[pl.ds(tok0, 16)]
            i1 = x.astype(jnp.int32)
            fr = x - i1.astype(jnp.float32)
            i2 = jnp.minimum(i1 + jnp.where(fr > 0, one_i, zero_i), cap)
            g1 = plsc.load_gather(idx_v, [i1])
            g2 = plsc.load_gather(idx_v, [i2])
            v1 = g1 - g1.astype(jnp.int32).astype(jnp.float32)
            v2 = 1.0 - (g2 - g2.astype(jnp.int32).astype(jnp.float32))
            eq = i1 == i2
            v1 = jnp.where(eq, one_f, v1)
            v2 = jnp.where(eq, one_f, v2)
            for c in range(4):
                bi = c % 2
                buf = bufs[bi]
                sem = sems[bi]
                dst = out_hbm.at[pl.ds(tok0 + c * _CH, _CH)]

                def _drain_and_clear(pc, buf=buf, sem=sem, dst=dst, bi=bi):
                    pltpu.make_async_copy(buf, dst, sem).wait()
                    pmask = (lane >> 2) == pc
                    plsc.store_scatter(buf, [rl, s1[bi, :]], zero16, mask=pmask)
                    plsc.store_scatter(buf, [rl, s2[bi, :]], zero16, mask=pmask)

                if c < 2:
                    @pl.when(b > 0)
                    def _():
                        _drain_and_clear(jnp.int32(c + 2))
                else:
                    _drain_and_clear(jnp.int32(c - 2))
                mask = (lane >> 2) == c
                plsc.store_scatter(buf, [rl, i1], v1, mask=mask)
                plsc.store_scatter(buf, [rl, i2], v2, mask=mask)
                s1[bi, :] = i1
                s2[bi, :] = i2
                pltpu.make_async_copy(buf, dst, sem).start()

        dst0 = out_hbm.at[pl.ds(base, _CH)]
        pltpu.make_async_copy(buf0, dst0, sem0).wait()
        pltpu.make_async_copy(buf1, dst0, sem1).wait()

    return k(index)


def kernel(index):
    return _sc_bandwidthify(index)


# speedup vs baseline: 3.6570x; 1.0099x over previous
import dataclasses
import functools

import jax
import jax.numpy as jnp
from jax import lax
from jax.experimental import pallas as pl
from jax.experimental.pallas import tpu as pltpu
from jax.experimental.pallas import tpu_sc as plsc

_B = 8192
_NW = 32
_RPW = _B // _NW
_CH = 4
_NBATCH = _RPW // 16


def _sc_compiler_params():
    cp = pltpu.CompilerParams()
    if "needs_layout_passes" in pltpu.CompilerParams.__dataclass_fields__:
        cp = dataclasses.replace(cp, needs_layout_passes=False)
    return cp


def _sc_bandwidthify(index):
    mesh = plsc.VectorSubcoreMesh(core_axis_name="c", subcore_axis_name="s")

    @functools.partial(
        pl.kernel,
        out_type=jax.ShapeDtypeStruct((_B, _B), jnp.float32),
        mesh=mesh,
        compiler_params=_sc_compiler_params(),
        scratch_types=[
            pltpu.VMEM((_B,), jnp.float32),
            pltpu.VMEM((_CH, _B), jnp.float32),
            pltpu.VMEM((_CH, _B), jnp.float32),
            pltpu.VMEM((2, 16), jnp.int32),
            pltpu.VMEM((2, 16), jnp.int32),
            pltpu.SemaphoreType.DMA,
            pltpu.SemaphoreType.DMA,
        ],
    )
    def k(idx_hbm, out_hbm, idx_v, buf0, buf1, s1, s2, sem0, sem1):
        wid = lax.axis_index("s") * 2 + lax.axis_index("c")
        base = wid * _RPW
        pltpu.sync_copy(idx_hbm, idx_v)
        zero16 = jnp.zeros((16,), jnp.float32)

        @pl.loop(0, _B // 16)
        def _(j):
            for r in range(_CH):
                buf0[r, pl.ds(j * 16, 16)] = zero16
                buf1[r, pl.ds(j * 16, 16)] = zero16

        lane = lax.iota(jnp.int32, 16)
        rl = lane & (_CH - 1)
        one_i = jnp.ones((16,), jnp.int32)
        zero_i = jnp.zeros((16,), jnp.int32)
        one_f = jnp.ones((16,), jnp.float32)
        cap = jnp.full((16,), _B - 1, jnp.int32)
        bufs = (buf0, buf1)
        sems = (sem0, sem1)

        @pl.loop(0, _NBATCH)
        def _(b):
            tok0 = base + b * 16
            x = idx_v[pl.ds(tok0, 16)]
            i1 = x.astype(jnp.int32)
            fr = x - i1.astype(jnp.float32)
            i2 = jnp.minimum(i1 + jnp.where(fr > 0, one_i, zero_i), cap)
            g1 = plsc.load_gather(idx_v, [i1])
            g2 = plsc.load_gather(idx_v, [i2])
            v1 = g1 - g1.astype(jnp.int32).astype(jnp.float32)
            v2 = 1.0 - (g2 - g2.astype(jnp.int32).astype(jnp.float32))
            eq = i1 == i2
            v1 = jnp.where(eq, one_f, v1)
            v2 = jnp.where(eq, one_f, v2)
            for c in range(4):
                bi = c % 2
                buf = bufs[bi]
                sem = sems[bi]
                dst = out_hbm.at[pl.ds(tok0 + c * _CH, _CH)]

                def _drain_and_clear(pc, buf=buf, sem=sem, dst=dst, bi=bi):
                    pltpu.make_async_copy(buf, dst, sem).wait()
                    pmask = (lane >> 2) == pc
                    plsc.store_scatter(buf, [rl, s1[bi, :]], zero16, mask=pmask)
                    plsc.store_scatter(buf, [rl, s2[bi, :]], zero16, mask=pmask)

                if c < 2:
                    @pl.when(b > 0)
                    def _():
                        _drain_and_clear(jnp.int32(c + 2))
                else:
                    _drain_and_clear(jnp.int32(c - 2))
                mask = (lane >> 2) == c
                plsc.store_scatter(buf, [rl, i1], v1, mask=mask)
                plsc.store_scatter(buf, [rl, i2], v2, mask=mask)
                s1[bi, :] = i1
                s2[bi, :] = i2
                pltpu.make_async_copy(buf, dst, sem).start()

        dst0 = out_hbm.at[pl.ds(base, _CH)]
        pltpu.make_async_copy(buf0, dst0, sem0).wait()
        pltpu.make_async_copy(buf1, dst0, sem1).wait()

    return k(index)


_CH8 = 8


def _sc_bandwidthify_t8(index):
    mesh = plsc.VectorSubcoreMesh(core_axis_name="c", subcore_axis_name="s")

    @functools.partial(
        pl.kernel,
        out_type=jax.ShapeDtypeStruct((_B, _B), jnp.float32),
        mesh=mesh,
        compiler_params=_sc_compiler_params(),
        scratch_types=[
            pltpu.VMEM((_B,), jnp.float32),
            pltpu.VMEM((_CH8, _B), jnp.float32),
            pltpu.SemaphoreType.DMA,
        ],
    )
    def k(idx_hbm, out_hbm, idx_v, buf, sem):
        wid = lax.axis_index("s") * 2 + lax.axis_index("c")
        base = wid * _RPW
        pltpu.sync_copy(idx_hbm, idx_v)
        zero16 = jnp.zeros((16,), jnp.float32)

        @pl.loop(0, _B // 16)
        def _(j):
            for r in range(_CH8):
                buf[r, pl.ds(j * 16, 16)] = zero16

        lane = lax.iota(jnp.int32, 16)
        rl = lane & (_CH8 - 1)
        one_i = jnp.ones((16,), jnp.int32)
        zero_i = jnp.zeros((16,), jnp.int32)
        one_f = jnp.ones((16,), jnp.float32)
        cap = jnp.full((16,), _B - 1, jnp.int32)

        @pl.loop(0, _NBATCH)
        def _(b):
            tok0 = base + b * 16
            x = idx_v[pl.ds(tok0, 16)]
            i1 = x.astype(jnp.int32)
            fr = x - i1.astype(jnp.float32)
            i2 = jnp.minimum(i1 + jnp.where(fr > 0, one_i, zero_i), cap)
            g1 = plsc.load_gather(idx_v, [i1])
            g2 = plsc.load_gather(idx_v, [i2])
            v1 = g1 - g1.astype(jnp.int32).astype(jnp.float32)
            v2 = 1.0 - (g2 - g2.astype(jnp.int32).astype(jnp.float32))
            eq = i1 == i2
            v1 = jnp.where(eq, one_f, v1)
            v2 = jnp.where(eq, one_f, v2)
            for c in range(2):
                mask = (lane >> 3) == c
                plsc.store_scatter(buf, [rl, i1], v1, mask=mask)
                plsc.store_scatter(buf, [rl, i2], v2, mask=mask)
                dst = out_hbm.at[pl.ds(tok0 + c * _CH8, _CH8)]
                cp = pltpu.make_async_copy(buf, dst, sem)
                cp.start()
                cp.wait()
                plsc.store_scatter(buf, [rl, i1], zero16, mask=mask)
                plsc.store_scatter(buf, [rl, i2], zero16, mask=mask)

    return k(index)


def kernel(index):
    return _sc_bandwidthify_t8(index)
